# Initial kernel scaffold; baseline (speedup 1.0000x reference)
import jax
import jax.numpy as jnp
from jax.experimental import pallas as pl


def kernel(x, edge_index, pos, W1, b1, W2, b2, W3, b3, Wlin, blin):
    raise NotImplementedError("write your pallas kernel here")


# speedup vs baseline: 4.8957x; 4.8957x over previous
import functools

import jax
import jax.numpy as jnp
from jax import lax
from jax.experimental import pallas as pl
from jax.experimental.pallas import tpu as pltpu
from jax.experimental.pallas import tpu_sc as plsc

N_NODES = 10000
N_EDGES = 320000
D_FEAT = 128
HIDDEN = 256
N_ACTIONS = 16

NC = 2
NS = 16
CH = 128

HALF = 5056
ACC_ROWS = 5120
LTRASH = HALF
RPT = ACC_ROWS // NS
STRIPE = (CH, CH, RPT - 2 * CH)
OUT_ROWS = 10240

EPT = 20224
NCHUNK = EPT // CH
E_PAD = NS * EPT
EPT_D = E_PAD // (NC * NS)
NCHUNK_D = EPT_D // CH


def _deg_body(dst_d, zeros16, ones16, deg_out, didx, ones_v, buf, acc):
    c = lax.axis_index("c")
    s = lax.axis_index("s")
    pltpu.sync_copy(ones16, ones_v)
    for g in range(2):
        pltpu.sync_copy(zeros16, buf)
        off = 0
        for sz in STRIPE:
            pltpu.sync_copy(buf.at[pl.ds(0, sz)],
                            acc.at[pl.ds(s * RPT + off, sz)])
            off += sz
        pltpu.sync_copy(dst_d.at[c, g, s], didx)
        plsc.subcore_barrier()

        def step(j, _):
            pltpu.sync_copy(ones_v, acc.at[didx.at[j]], add=True)
            return 0

        lax.fori_loop(0, NCHUNK_D, step, 0)
        plsc.subcore_barrier()
        off = 0
        for sz in STRIPE:
            r0 = s * RPT + off
            pltpu.sync_copy(acc.at[pl.ds(r0, sz)], buf.at[pl.ds(0, sz)])
            pltpu.sync_copy(buf.at[pl.ds(0, sz)],
                            deg_out.at[c, pl.ds(g * HALF + r0, sz)])
            off += sz


_deg_kernel = functools.partial(
    pl.kernel,
    out_type=jax.ShapeDtypeStruct((NC, OUT_ROWS, 16), jnp.float32),
    mesh=plsc.VectorSubcoreMesh(core_axis_name="c", subcore_axis_name="s"),
    scratch_types=[
        pltpu.VMEM((NCHUNK_D, CH), jnp.int32),
        pltpu.VMEM((CH, 16), jnp.float32),
        pltpu.VMEM((CH, 16), jnp.float32),
        pltpu.VMEM_SHARED((ACC_ROWS, 16), jnp.float32),
    ],
)(_deg_body)


def _scatter_body(hp, src_g, dst_gl, zeros128, out, sidx, didx, rows0, rows1,
                  sem0, sem1, acc):
    c = lax.axis_index("c")
    s = lax.axis_index("s")
    pltpu.sync_copy(src_g.at[c, s], sidx)
    for g in range(2):
        pltpu.sync_copy(zeros128, rows0)
        off = 0
        for sz in STRIPE:
            pltpu.sync_copy(rows0.at[pl.ds(0, sz)],
                            acc.at[pl.ds(s * RPT + off, sz)])
            off += sz
        pltpu.sync_copy(dst_gl.at[g, s], didx)
        plsc.subcore_barrier()

        def step(i, _):
            j0 = 2 * i
            j1 = 2 * i + 1
            cp0 = pltpu.async_copy(hp.at[sidx.at[j0]], rows0, sem0)
            cp1 = pltpu.async_copy(hp.at[sidx.at[j1]], rows1, sem1)
            cp0.wait()
            pltpu.sync_copy(rows0, acc.at[didx.at[j0]], add=True)
            cp1.wait()
            pltpu.sync_copy(rows1, acc.at[didx.at[j1]], add=True)
            return 0

        lax.fori_loop(0, NCHUNK // 2, step, 0)
        plsc.subcore_barrier()
        off = 0
        for sz in STRIPE:
            r0 = s * RPT + off
            pltpu.sync_copy(acc.at[pl.ds(r0, sz)], rows0.at[pl.ds(0, sz)])
            pltpu.sync_copy(rows0.at[pl.ds(0, sz)],
                            out.at[pl.ds(c * OUT_ROWS + g * HALF + r0, sz)])
            off += sz


_scatter_kernel = functools.partial(
    pl.kernel,
    out_type=jax.ShapeDtypeStruct((NC * OUT_ROWS, D_FEAT), jnp.float32),
    mesh=plsc.VectorSubcoreMesh(core_axis_name="c", subcore_axis_name="s"),
    scratch_types=[
        pltpu.VMEM((NCHUNK, CH), jnp.int32),
        pltpu.VMEM((NCHUNK, CH), jnp.int32),
        pltpu.VMEM((CH, D_FEAT), jnp.float32),
        pltpu.VMEM((CH, D_FEAT), jnp.float32),
        pltpu.SemaphoreType.DMA,
        pltpu.SemaphoreType.DMA,
        pltpu.VMEM_SHARED((ACC_ROWS, D_FEAT), jnp.float32),
    ],
)(_scatter_body)


BN = 1000
GRID = N_NODES // BN


def _prep_body(x_ref, w_ref, degp_ref, hp_ref, dis_ref):
    deg = degp_ref[0] + degp_ref[1] + 1.0
    dis = lax.rsqrt(deg)
    dis_ref[...] = dis
    h = jnp.dot(x_ref[...], w_ref[...], preferred_element_type=jnp.float32)
    hp = h * dis[:, :1]
    hp_ref[0] = hp[:, :D_FEAT]
    hp_ref[1] = hp[:, D_FEAT:]


_prep_call = pl.pallas_call(
    _prep_body,
    grid=(GRID,),
    in_specs=[
        pl.BlockSpec((BN, D_FEAT), lambda i: (i, 0)),
        pl.BlockSpec((D_FEAT, HIDDEN), lambda i: (0, 0)),
        pl.BlockSpec((NC, BN, 16), lambda i: (0, i, 0)),
    ],
    out_specs=[
        pl.BlockSpec((NC, BN, D_FEAT), lambda i: (0, i, 0)),
        pl.BlockSpec((BN, 16), lambda i: (i, 0)),
    ],
    out_shape=[
        jax.ShapeDtypeStruct((NC, N_NODES, D_FEAT), jnp.float32),
        jax.ShapeDtypeStruct((N_NODES, 16), jnp.float32),
    ],
)


def _mid_body(osc_ref, hp_ref, dis_ref, b_ref, w_ref, out_ref):
    disc = dis_ref[:, :1]
    hp = jnp.concatenate([hp_ref[0], hp_ref[1]], axis=1)
    m = jnp.concatenate([osc_ref[0], osc_ref[1]], axis=1) + hp
    z = jnp.maximum(disc * m + b_ref[...], 0.0)
    h = jnp.dot(z, w_ref[...], preferred_element_type=jnp.float32)
    hpn = h * disc
    out_ref[0] = hpn[:, :D_FEAT]
    out_ref[1] = hpn[:, D_FEAT:]


_mid_call = pl.pallas_call(
    _mid_body,
    grid=(GRID,),
    in_specs=[
        pl.BlockSpec((NC, BN, D_FEAT), lambda i: (0, i, 0)),
        pl.BlockSpec((NC, BN, D_FEAT), lambda i: (0, i, 0)),
        pl.BlockSpec((BN, 16), lambda i: (i, 0)),
        pl.BlockSpec((1, HIDDEN), lambda i: (0, 0)),
        pl.BlockSpec((HIDDEN, HIDDEN), lambda i: (0, 0)),
    ],
    out_specs=pl.BlockSpec((NC, BN, D_FEAT), lambda i: (0, i, 0)),
    out_shape=jax.ShapeDtypeStruct((NC, N_NODES, D_FEAT), jnp.float32),
)


def _fin_body(osc_ref, hp_ref, dis_ref, b_ref, wlin_ref, blin_ref, out_ref,
              sum_ref, max_ref):
    i = pl.program_id(0)
    disc = dis_ref[:, :1]
    hp = jnp.concatenate([hp_ref[0], hp_ref[1]], axis=1)
    m = jnp.concatenate([osc_ref[0], osc_ref[1]], axis=1) + hp
    z = jnp.maximum(disc * m + b_ref[...], 0.0)
    zs = jnp.sum(z, axis=0, keepdims=True)
    zm = jnp.max(z, axis=0, keepdims=True)

    @pl.when(i == 0)
    def _():
        sum_ref[...] = zs
        max_ref[...] = zm

    @pl.when(i > 0)
    def _():
        sum_ref[...] = sum_ref[...] + zs
        max_ref[...] = jnp.maximum(max_ref[...], zm)

    g = jnp.concatenate(
        [sum_ref[...] * (1.0 / N_NODES), max_ref[...], sum_ref[...]], axis=1)
    out_ref[...] = (
        jnp.dot(g, wlin_ref[...], preferred_element_type=jnp.float32)
        + blin_ref[...])


_fin_call = pl.pallas_call(
    _fin_body,
    grid=(GRID,),
    in_specs=[
        pl.BlockSpec((NC, BN, D_FEAT), lambda i: (0, i, 0)),
        pl.BlockSpec((NC, BN, D_FEAT), lambda i: (0, i, 0)),
        pl.BlockSpec((BN, 16), lambda i: (i, 0)),
        pl.BlockSpec((1, HIDDEN), lambda i: (0, 0)),
        pl.BlockSpec((HIDDEN * 3, N_ACTIONS), lambda i: (0, 0)),
        pl.BlockSpec((1, N_ACTIONS), lambda i: (0, 0)),
    ],
    out_specs=pl.BlockSpec((1, N_ACTIONS), lambda i: (0, 0)),
    out_shape=jax.ShapeDtypeStruct((1, N_ACTIONS), jnp.float32),
    scratch_shapes=[
        pltpu.VMEM((1, HIDDEN), jnp.float32),
        pltpu.VMEM((1, HIDDEN), jnp.float32),
    ],
)


def kernel(x, edge_index, pos, W1, b1, W2, b2, W3, b3, Wlin, blin):
    src = edge_index[0].astype(jnp.int32)
    dst = edge_index[1].astype(jnp.int32)
    pad = E_PAD - N_EDGES
    srcp = jnp.concatenate([src, jnp.zeros((pad,), jnp.int32)])
    dstp = jnp.concatenate([dst, jnp.full((pad,), 2 * HALF, jnp.int32)])
    src_r = srcp.reshape(NS, NCHUNK, CH)
    src_g = jnp.stack([src_r, src_r + N_NODES])
    dst_locs = []
    for g in range(2):
        loc = dstp - g * HALF
        loc = jnp.where((loc >= 0) & (loc < HALF), loc, LTRASH)
        dst_locs.append(loc)
    dst_gl = jnp.stack([l.reshape(NS, NCHUNK, CH) for l in dst_locs])
    dst_d = jnp.stack([l.reshape(NC, NS, NCHUNK_D, CH) for l in dst_locs],
                      axis=1)

    zeros16 = jnp.zeros((CH, 16), jnp.float32)
    ones16 = jnp.ones((CH, 16), jnp.float32)
    zeros128 = jnp.zeros((CH, D_FEAT), jnp.float32)
    b1r = b1.reshape(1, HIDDEN)
    b2r = b2.reshape(1, HIDDEN)
    b3r = b3.reshape(1, HIDDEN)
    blinr = blin.reshape(1, N_ACTIONS)

    degp = _deg_kernel(dst_d, zeros16, ones16)
    hp1, dis = _prep_call(x, W1, degp[:, :N_NODES])
    osc1 = _scatter_kernel(hp1.reshape(NC * N_NODES, D_FEAT),
                           src_g, dst_gl, zeros128)
    hp2 = _mid_call(osc1.reshape(NC, OUT_ROWS, D_FEAT), hp1, dis, b1r, W2)
    osc2 = _scatter_kernel(hp2.reshape(NC * N_NODES, D_FEAT),
                           src_g, dst_gl, zeros128)
    hp3 = _mid_call(osc2.reshape(NC, OUT_ROWS, D_FEAT), hp2, dis, b2r, W3)
    osc3 = _scatter_kernel(hp3.reshape(NC * N_NODES, D_FEAT),
                           src_g, dst_gl, zeros128)
    return _fin_call(osc3.reshape(NC, OUT_ROWS, D_FEAT), hp3, dis, b3r,
                     Wlin, blinr)


# speedup vs baseline: 5.0172x; 1.0248x over previous
import functools

import jax
import jax.numpy as jnp
from jax import lax
from jax.experimental import pallas as pl
from jax.experimental.pallas import tpu as pltpu
from jax.experimental.pallas import tpu_sc as plsc

N_NODES = 10000
N_EDGES = 320000
D_FEAT = 128
HIDDEN = 256
N_ACTIONS = 16

NC = 2
NS = 16
CH = 128

HALF = 5056
ACC_ROWS = 5120
LTRASH = HALF
RPT = ACC_ROWS // NS
STRIPE = (CH, CH, RPT - 2 * CH)
OUT_ROWS = 10240

EPT = 20224
NCHUNK = EPT // CH
E_PAD = NS * EPT
EPT_D = E_PAD // (NC * NS)
NCHUNK_D = EPT_D // CH


DEG_ROWS = 10240
DPT = DEG_ROWS // NS


def _deg_body(dst_d, zeros1, ones1, deg_out, didx, ones_v, buf, acc):
    c = lax.axis_index("c")
    s = lax.axis_index("s")
    pltpu.sync_copy(ones1, ones_v)
    pltpu.sync_copy(zeros1, buf)
    pltpu.sync_copy(buf, acc.at[pl.ds(s * DPT, DPT)])
    pltpu.sync_copy(dst_d.at[c, s], didx)
    plsc.subcore_barrier()

    def step(j, _):
        pltpu.sync_copy(ones_v, acc.at[didx.at[j]], add=True)
        return 0

    lax.fori_loop(0, NCHUNK_D, step, 0)
    plsc.subcore_barrier()
    pltpu.sync_copy(acc.at[pl.ds(s * DPT, DPT)], buf)
    pltpu.sync_copy(buf, deg_out.at[c, pl.ds(s * DPT, DPT)])


_deg_kernel = functools.partial(
    pl.kernel,
    out_type=jax.ShapeDtypeStruct((NC, DEG_ROWS), jnp.float32),
    mesh=plsc.VectorSubcoreMesh(core_axis_name="c", subcore_axis_name="s"),
    scratch_types=[
        pltpu.VMEM((NCHUNK_D, CH), jnp.int32),
        pltpu.VMEM((CH,), jnp.float32),
        pltpu.VMEM((DPT,), jnp.float32),
        pltpu.VMEM_SHARED((DEG_ROWS,), jnp.float32),
    ],
)(_deg_body)


def _scatter_body(hp, src_g, dst_gl, zeros128, out, sidx, didx, rows0, rows1,
                  sem0, sem1, acc):
    c = lax.axis_index("c")
    s = lax.axis_index("s")
    pltpu.sync_copy(src_g.at[c, s], sidx)
    for g in range(2):
        pltpu.sync_copy(zeros128, rows0)
        off = 0
        for sz in STRIPE:
            pltpu.sync_copy(rows0.at[pl.ds(0, sz)],
                            acc.at[pl.ds(s * RPT + off, sz)])
            off += sz
        pltpu.sync_copy(dst_gl.at[g, s], didx)
        plsc.subcore_barrier()

        def step(i, _):
            j0 = 2 * i
            j1 = 2 * i + 1
            cp0 = pltpu.async_copy(hp.at[sidx.at[j0]], rows0, sem0)
            cp1 = pltpu.async_copy(hp.at[sidx.at[j1]], rows1, sem1)
            cp0.wait()
            pltpu.sync_copy(rows0, acc.at[didx.at[j0]], add=True)
            cp1.wait()
            pltpu.sync_copy(rows1, acc.at[didx.at[j1]], add=True)
            return 0

        lax.fori_loop(0, NCHUNK // 2, step, 0)
        plsc.subcore_barrier()
        off = 0
        for sz in STRIPE:
            r0 = s * RPT + off
            pltpu.sync_copy(acc.at[pl.ds(r0, sz)], rows0.at[pl.ds(0, sz)])
            pltpu.sync_copy(rows0.at[pl.ds(0, sz)],
                            out.at[pl.ds(c * OUT_ROWS + g * HALF + r0, sz)])
            off += sz


_scatter_kernel = functools.partial(
    pl.kernel,
    out_type=jax.ShapeDtypeStruct((NC * OUT_ROWS, D_FEAT), jnp.float32),
    mesh=plsc.VectorSubcoreMesh(core_axis_name="c", subcore_axis_name="s"),
    scratch_types=[
        pltpu.VMEM((NCHUNK, CH), jnp.int32),
        pltpu.VMEM((NCHUNK, CH), jnp.int32),
        pltpu.VMEM((CH, D_FEAT), jnp.float32),
        pltpu.VMEM((CH, D_FEAT), jnp.float32),
        pltpu.SemaphoreType.DMA,
        pltpu.SemaphoreType.DMA,
        pltpu.VMEM_SHARED((ACC_ROWS, D_FEAT), jnp.float32),
    ],
)(_scatter_body)


BN = 1000
GRID = N_NODES // BN


def _prep_body(x_ref, w_ref, degp_ref, hp_ref, dis_ref):
    deg = degp_ref[0] + degp_ref[1] + 1.0
    dis = lax.rsqrt(deg)
    dis_ref[...] = dis
    h = jnp.dot(x_ref[...], w_ref[...], preferred_element_type=jnp.float32)
    hp = h * dis
    hp_ref[0] = hp[:, :D_FEAT]
    hp_ref[1] = hp[:, D_FEAT:]


_prep_call = pl.pallas_call(
    _prep_body,
    grid=(GRID,),
    in_specs=[
        pl.BlockSpec((BN, D_FEAT), lambda i: (i, 0)),
        pl.BlockSpec((D_FEAT, HIDDEN), lambda i: (0, 0)),
        pl.BlockSpec((NC, BN, 1), lambda i: (0, i, 0)),
    ],
    out_specs=[
        pl.BlockSpec((NC, BN, D_FEAT), lambda i: (0, i, 0)),
        pl.BlockSpec((BN, 1), lambda i: (i, 0)),
    ],
    out_shape=[
        jax.ShapeDtypeStruct((NC, N_NODES, D_FEAT), jnp.float32),
        jax.ShapeDtypeStruct((N_NODES, 1), jnp.float32),
    ],
)


def _mid_body(osc_ref, hp_ref, dis_ref, b_ref, w_ref, out_ref):
    disc = dis_ref[...]
    hp = jnp.concatenate([hp_ref[0], hp_ref[1]], axis=1)
    m = jnp.concatenate([osc_ref[0], osc_ref[1]], axis=1) + hp
    z = jnp.maximum(disc * m + b_ref[...], 0.0)
    h = jnp.dot(z, w_ref[...], preferred_element_type=jnp.float32)
    hpn = h * disc
    out_ref[0] = hpn[:, :D_FEAT]
    out_ref[1] = hpn[:, D_FEAT:]


_mid_call = pl.pallas_call(
    _mid_body,
    grid=(GRID,),
    in_specs=[
        pl.BlockSpec((NC, BN, D_FEAT), lambda i: (0, i, 0)),
        pl.BlockSpec((NC, BN, D_FEAT), lambda i: (0, i, 0)),
        pl.BlockSpec((BN, 1), lambda i: (i, 0)),
        pl.BlockSpec((1, HIDDEN), lambda i: (0, 0)),
        pl.BlockSpec((HIDDEN, HIDDEN), lambda i: (0, 0)),
    ],
    out_specs=pl.BlockSpec((NC, BN, D_FEAT), lambda i: (0, i, 0)),
    out_shape=jax.ShapeDtypeStruct((NC, N_NODES, D_FEAT), jnp.float32),
)


def _fin_body(osc_ref, hp_ref, dis_ref, b_ref, wlin_ref, blin_ref, out_ref,
              sum_ref, max_ref):
    i = pl.program_id(0)
    disc = dis_ref[...]
    hp = jnp.concatenate([hp_ref[0], hp_ref[1]], axis=1)
    m = jnp.concatenate([osc_ref[0], osc_ref[1]], axis=1) + hp
    z = jnp.maximum(disc * m + b_ref[...], 0.0)
    zs = jnp.sum(z, axis=0, keepdims=True)
    zm = jnp.max(z, axis=0, keepdims=True)

    @pl.when(i == 0)
    def _():
        sum_ref[...] = zs
        max_ref[...] = zm

    @pl.when(i > 0)
    def _():
        sum_ref[...] = sum_ref[...] + zs
        max_ref[...] = jnp.maximum(max_ref[...], zm)

    g = jnp.concatenate(
        [sum_ref[...] * (1.0 / N_NODES), max_ref[...], sum_ref[...]], axis=1)
    out_ref[...] = (
        jnp.dot(g, wlin_ref[...], preferred_element_type=jnp.float32)
        + blin_ref[...])


_fin_call = pl.pallas_call(
    _fin_body,
    grid=(GRID,),
    in_specs=[
        pl.BlockSpec((NC, BN, D_FEAT), lambda i: (0, i, 0)),
        pl.BlockSpec((NC, BN, D_FEAT), lambda i: (0, i, 0)),
        pl.BlockSpec((BN, 1), lambda i: (i, 0)),
        pl.BlockSpec((1, HIDDEN), lambda i: (0, 0)),
        pl.BlockSpec((HIDDEN * 3, N_ACTIONS), lambda i: (0, 0)),
        pl.BlockSpec((1, N_ACTIONS), lambda i: (0, 0)),
    ],
    out_specs=pl.BlockSpec((1, N_ACTIONS), lambda i: (0, 0)),
    out_shape=jax.ShapeDtypeStruct((1, N_ACTIONS), jnp.float32),
    scratch_shapes=[
        pltpu.VMEM((1, HIDDEN), jnp.float32),
        pltpu.VMEM((1, HIDDEN), jnp.float32),
    ],
)


def kernel(x, edge_index, pos, W1, b1, W2, b2, W3, b3, Wlin, blin):
    src = edge_index[0].astype(jnp.int32)
    dst = edge_index[1].astype(jnp.int32)
    pad = E_PAD - N_EDGES
    srcp = jnp.concatenate([src, jnp.zeros((pad,), jnp.int32)])
    dstp = jnp.concatenate([dst, jnp.full((pad,), 2 * HALF, jnp.int32)])
    src_r = srcp.reshape(NS, NCHUNK, CH)
    src_g = jnp.stack([src_r, src_r + N_NODES])
    dst_locs = []
    for g in range(2):
        loc = dstp - g * HALF
        loc = jnp.where((loc >= 0) & (loc < HALF), loc, LTRASH)
        dst_locs.append(loc)
    dst_gl = jnp.stack([l.reshape(NS, NCHUNK, CH) for l in dst_locs])
    dst_d = dstp.reshape(NC, NS, NCHUNK_D, CH)

    zeros1 = jnp.zeros((DPT,), jnp.float32)
    ones1 = jnp.ones((CH,), jnp.float32)
    zeros128 = jnp.zeros((CH, D_FEAT), jnp.float32)
    b1r = b1.reshape(1, HIDDEN)
    b2r = b2.reshape(1, HIDDEN)
    b3r = b3.reshape(1, HIDDEN)
    blinr = blin.reshape(1, N_ACTIONS)

    degp = _deg_kernel(dst_d, zeros1, ones1)
    hp1, dis = _prep_call(x, W1, degp.reshape(NC, DEG_ROWS, 1))
    osc1 = _scatter_kernel(hp1.reshape(NC * N_NODES, D_FEAT),
                           src_g, dst_gl, zeros128)
    hp2 = _mid_call(osc1.reshape(NC, OUT_ROWS, D_FEAT), hp1, dis, b1r, W2)
    osc2 = _scatter_kernel(hp2.reshape(NC * N_NODES, D_FEAT),
                           src_g, dst_gl, zeros128)
    hp3 = _mid_call(osc2.reshape(NC, OUT_ROWS, D_FEAT), hp2, dis, b2r, W3)
    osc3 = _scatter_kernel(hp3.reshape(NC * N_NODES, D_FEAT),
                           src_g, dst_gl, zeros128)
    return _fin_call(osc3.reshape(NC, OUT_ROWS, D_FEAT), hp3, dis, b3r,
                     Wlin, blinr)
